# Initial kernel scaffold; baseline (speedup 1.0000x reference)
import jax
import jax.numpy as jnp
from jax.experimental import pallas as pl


def kernel(x, weight, bias, gamma, beta):
    raise NotImplementedError("write your pallas kernel here")


# speedup vs baseline: 1.1779x; 1.1779x over previous
import functools

import jax
import jax.numpy as jnp
from jax.experimental import pallas as pl
from jax.experimental.pallas import tpu as pltpu


def _conv_relu_stats_kernel(x_ref, w_ref, b_ref, mask_ref, y_ref, s_ref, q_ref,
                            p_ref, *, cin, kh, kw, wp, span):
    img = x_ref[0]
    for t in range(kh * kw):
        ki, kj = divmod(t, kw)
        start = ki * wp + kj
        p_ref[t * cin:(t + 1) * cin, :] = img[:, start:start + span]

    acc = jax.lax.dot_general(w_ref[...], p_ref[...],
                              dimension_numbers=(((1,), (0,)), ((), ())),
                              preferred_element_type=jnp.float32)
    y = jnp.maximum(acc + b_ref[...], 0.0)
    y_ref[...] = y.astype(jnp.bfloat16)[None]

    ys = y * mask_ref[...]
    s_ref[...] = jnp.sum(ys, axis=1, keepdims=True)[None]
    q_ref[...] = jnp.sum(ys * ys, axis=1, keepdims=True)[None]


def _bn_apply_kernel(y_ref, sc_ref, sh_ref, o_ref, *, ow):
    y = y_ref[0][:, :, :ow].astype(jnp.float32)
    sc = sc_ref[...][:, :, None]
    sh = sh_ref[...][:, :, None]
    o_ref[...] = (y * sc + sh)[None]


def kernel(x, weight, bias, gamma, beta):
    stride, padding, eps = 1, 1, 1e-5
    N, Cin, H, W = x.shape
    Cout, _, KH, KW = weight.shape

    OH = (H + 2 * padding - KH) // stride + 1
    OW = (W + 2 * padding - KW) // stride + 1
    Hp, Wp = H + 2 * padding, W + 2 * padding
    span = OH * Wp
    M = N * OH * OW
    K = KH * KW * Cin

    xp = jnp.pad(x, ((0, 0), (0, 0), (padding, padding),
                     (padding, padding))).astype(jnp.bfloat16)
    xflat = xp.reshape(N, Cin, Hp * Wp)
    L = (KH - 1) * Wp + (KW - 1) + span
    if L > Hp * Wp:
        xflat = jnp.pad(xflat, ((0, 0), (0, 0), (0, L - Hp * Wp)))
    L = max(L, Hp * Wp)

    w2 = jnp.transpose(weight, (0, 2, 3, 1)).reshape(Cout, K).astype(jnp.bfloat16)
    b2 = bias.reshape(Cout, 1).astype(jnp.float32)
    valid = (jnp.arange(span, dtype=jnp.int32) % Wp < OW).astype(
        jnp.float32).reshape(1, span)

    cparams = pltpu.CompilerParams(
        dimension_semantics=("parallel",),
        vmem_limit_bytes=48 * 1024 * 1024)

    k1 = functools.partial(_conv_relu_stats_kernel,
                           cin=Cin, kh=KH, kw=KW, wp=Wp, span=span)
    y_flat, part_s, part_q = pl.pallas_call(
        k1,
        out_shape=(jax.ShapeDtypeStruct((N, Cout, span), jnp.bfloat16),
                   jax.ShapeDtypeStruct((N, Cout, 1), jnp.float32),
                   jax.ShapeDtypeStruct((N, Cout, 1), jnp.float32)),
        grid_spec=pltpu.PrefetchScalarGridSpec(
            num_scalar_prefetch=0,
            grid=(N,),
            in_specs=[pl.BlockSpec((1, Cin, L), lambda n: (n, 0, 0)),
                      pl.BlockSpec((Cout, K), lambda n: (0, 0)),
                      pl.BlockSpec((Cout, 1), lambda n: (0, 0)),
                      pl.BlockSpec((1, span), lambda n: (0, 0))],
            out_specs=(pl.BlockSpec((1, Cout, span), lambda n: (n, 0, 0)),
                       pl.BlockSpec((1, Cout, 1), lambda n: (n, 0, 0)),
                       pl.BlockSpec((1, Cout, 1), lambda n: (n, 0, 0))),
            scratch_shapes=[pltpu.VMEM((K, span), jnp.bfloat16)]),
        compiler_params=cparams,
    )(xflat, w2, b2, valid)

    s = jnp.sum(part_s[:, :, 0], axis=0)
    q = jnp.sum(part_q[:, :, 0], axis=0)
    mean = s / M
    var = jnp.maximum(q / M - mean * mean, 0.0)
    inv = jax.lax.rsqrt(var + eps)
    g = gamma.astype(jnp.float32)
    scale = (g * inv).reshape(Cout, 1)
    shift = (beta.astype(jnp.float32) - mean * g * inv).reshape(Cout, 1)

    y4 = y_flat.reshape(N, Cout, OH, Wp)
    out = pl.pallas_call(
        functools.partial(_bn_apply_kernel, ow=OW),
        out_shape=jax.ShapeDtypeStruct((N, Cout, OH, OW), jnp.float32),
        grid_spec=pltpu.PrefetchScalarGridSpec(
            num_scalar_prefetch=0,
            grid=(N,),
            in_specs=[pl.BlockSpec((1, Cout, OH, Wp), lambda n: (n, 0, 0, 0)),
                      pl.BlockSpec((Cout, 1), lambda n: (0, 0)),
                      pl.BlockSpec((Cout, 1), lambda n: (0, 0))],
            out_specs=pl.BlockSpec((1, Cout, OH, OW), lambda n: (n, 0, 0, 0))),
        compiler_params=cparams,
    )(y4, scale, shift)

    return out


# speedup vs baseline: 1.9307x; 1.6391x over previous
import functools

import jax
import jax.numpy as jnp
from jax.experimental import pallas as pl
from jax.experimental.pallas import tpu as pltpu


def _conv_relu_stats_kernel(x_ref, w_ref, b_ref, y_ref, s_ref, q_ref,
                            img_ref, p_ref, *, cin, kh, kw, h, w, wp, span, ow):
    img_ref[...] = jnp.zeros_like(img_ref)
    xin = x_ref[0]
    for r in range(h):
        img_ref[:, (r + 1) * wp + 1:(r + 1) * wp + 1 + w] = (
            xin[:, r * w:(r + 1) * w].astype(jnp.bfloat16))

    img = img_ref[...]
    for t in range(kh * kw):
        ki, kj = divmod(t, kw)
        start = ki * wp + kj
        p_ref[t * cin:(t + 1) * cin, :] = img[:, start:start + span]

    acc = jax.lax.dot_general(w_ref[...], p_ref[...],
                              dimension_numbers=(((1,), (0,)), ((), ())),
                              preferred_element_type=jnp.float32)
    y = jnp.maximum(acc + b_ref[...], 0.0)

    oh = span // wp
    yc = jnp.concatenate([y[:, r * wp:r * wp + ow] for r in range(oh)], axis=1)
    y_ref[...] = yc.astype(jnp.bfloat16)[None]

    s_ref[...] = jnp.sum(yc, axis=1, keepdims=True)[None]
    q_ref[...] = jnp.sum(yc * yc, axis=1, keepdims=True)[None]


def _bn_apply_kernel(y_ref, sc_ref, sh_ref, o_ref):
    y = y_ref[0].astype(jnp.float32)
    o_ref[...] = (y * sc_ref[...] + sh_ref[...])[None]


def kernel(x, weight, bias, gamma, beta):
    stride, padding, eps = 1, 1, 1e-5
    del stride
    N, Cin, H, W = x.shape
    Cout, _, KH, KW = weight.shape

    OH = H + 2 * padding - KH + 1
    OW = W + 2 * padding - KW + 1
    Hp, Wp = H + 2 * padding, W + 2 * padding
    span = OH * Wp
    M = N * OH * OW
    K = KH * KW * Cin
    L = max((KH - 1) * Wp + (KW - 1) + span, Hp * Wp)

    xflat = x.reshape(N, Cin, H * W)

    w2 = jnp.transpose(weight, (0, 2, 3, 1)).reshape(Cout, K).astype(jnp.bfloat16)
    b2 = bias.reshape(Cout, 1).astype(jnp.float32)

    cparams = pltpu.CompilerParams(
        dimension_semantics=("parallel",),
        vmem_limit_bytes=48 * 1024 * 1024)

    k1 = functools.partial(_conv_relu_stats_kernel, cin=Cin, kh=KH, kw=KW,
                           h=H, w=W, wp=Wp, span=span, ow=OW)
    y_flat, part_s, part_q = pl.pallas_call(
        k1,
        out_shape=(jax.ShapeDtypeStruct((N, Cout, OH * OW), jnp.bfloat16),
                   jax.ShapeDtypeStruct((N, Cout, 1), jnp.float32),
                   jax.ShapeDtypeStruct((N, Cout, 1), jnp.float32)),
        grid_spec=pltpu.PrefetchScalarGridSpec(
            num_scalar_prefetch=0,
            grid=(N,),
            in_specs=[pl.BlockSpec((1, Cin, H * W), lambda n: (n, 0, 0)),
                      pl.BlockSpec((Cout, K), lambda n: (0, 0)),
                      pl.BlockSpec((Cout, 1), lambda n: (0, 0))],
            out_specs=(pl.BlockSpec((1, Cout, OH * OW), lambda n: (n, 0, 0)),
                       pl.BlockSpec((1, Cout, 1), lambda n: (n, 0, 0)),
                       pl.BlockSpec((1, Cout, 1), lambda n: (n, 0, 0))),
            scratch_shapes=[pltpu.VMEM((Cin, L), jnp.bfloat16),
                            pltpu.VMEM((K, span), jnp.bfloat16)]),
        compiler_params=cparams,
    )(xflat, w2, b2)

    s = jnp.sum(part_s[:, :, 0], axis=0)
    q = jnp.sum(part_q[:, :, 0], axis=0)
    mean = s / M
    var = jnp.maximum(q / M - mean * mean, 0.0)
    inv = jax.lax.rsqrt(var + eps)
    g = gamma.astype(jnp.float32)
    scale = (g * inv).reshape(Cout, 1)
    shift = (beta.astype(jnp.float32) - mean * g * inv).reshape(Cout, 1)

    out_flat = pl.pallas_call(
        _bn_apply_kernel,
        out_shape=jax.ShapeDtypeStruct((N, Cout, OH * OW), jnp.float32),
        grid_spec=pltpu.PrefetchScalarGridSpec(
            num_scalar_prefetch=0,
            grid=(N,),
            in_specs=[pl.BlockSpec((1, Cout, OH * OW), lambda n: (n, 0, 0)),
                      pl.BlockSpec((Cout, 1), lambda n: (0, 0)),
                      pl.BlockSpec((Cout, 1), lambda n: (0, 0))],
            out_specs=pl.BlockSpec((1, Cout, OH * OW), lambda n: (n, 0, 0))),
        compiler_params=cparams,
    )(y_flat, scale, shift)

    return out_flat.reshape(N, Cout, OH, OW)


# speedup vs baseline: 2.0381x; 1.0557x over previous
import functools

import jax
import jax.numpy as jnp
from jax.experimental import pallas as pl
from jax.experimental.pallas import tpu as pltpu


def _conv_relu_stats_kernel(x_ref, w_ref, b_ref, y_ref, s_ref, q_ref,
                            img_ref, p_ref, *, nb, cin, kh, kw, h, w, wp,
                            span, ow):
    oh = span // wp
    for b in range(nb):
        img_ref[...] = jnp.zeros_like(img_ref)
        xin = x_ref[b]
        for r in range(h):
            img_ref[:, (r + 1) * wp + 1:(r + 1) * wp + 1 + w] = (
                xin[:, r * w:(r + 1) * w].astype(jnp.bfloat16))

        img = img_ref[...]
        for t in range(kh * kw):
            ki, kj = divmod(t, kw)
            start = ki * wp + kj
            p_ref[t * cin:(t + 1) * cin, :] = img[:, start:start + span]

        acc = jax.lax.dot_general(w_ref[...], p_ref[...],
                                  dimension_numbers=(((1,), (0,)), ((), ())),
                                  preferred_element_type=jnp.float32)
        y = jnp.maximum(acc + b_ref[...], 0.0)

        yc = jnp.concatenate([y[:, r * wp:r * wp + ow] for r in range(oh)],
                             axis=1)
        y_ref[b] = yc.astype(jnp.bfloat16)

        s_ref[b] = jnp.sum(yc, axis=1, keepdims=True)
        q_ref[b] = jnp.sum(yc * yc, axis=1, keepdims=True)


def _bn_apply_kernel(y_ref, s_ref, q_ref, g_ref, be_ref, o_ref, *,
                     m, eps, nb):
    s = jnp.sum(s_ref[...], axis=0)
    q = jnp.sum(q_ref[...], axis=0)
    mean = s / m
    var = jnp.maximum(q / m - mean * mean, 0.0)
    inv = jax.lax.rsqrt(var + eps)
    scale = g_ref[...] * inv
    shift = be_ref[...] - mean * scale
    for b in range(nb):
        y = y_ref[b].astype(jnp.float32)
        o_ref[b] = y * scale + shift


def kernel(x, weight, bias, gamma, beta):
    padding, eps = 1, 1e-5
    N, Cin, H, W = x.shape
    Cout, _, KH, KW = weight.shape

    OH = H + 2 * padding - KH + 1
    OW = W + 2 * padding - KW + 1
    Hp, Wp = H + 2 * padding, W + 2 * padding
    span = OH * Wp
    M = N * OH * OW
    K = KH * KW * Cin
    L = max((KH - 1) * Wp + (KW - 1) + span, Hp * Wp)

    NB1 = 2 if N % 2 == 0 else 1
    NB2 = 4 if N % 4 == 0 else 1

    xflat = x.reshape(N, Cin, H * W)

    w2 = jnp.transpose(weight, (0, 2, 3, 1)).reshape(Cout, K).astype(jnp.bfloat16)
    b2 = bias.reshape(Cout, 1).astype(jnp.float32)
    g2 = gamma.reshape(Cout, 1).astype(jnp.float32)
    be2 = beta.reshape(Cout, 1).astype(jnp.float32)

    cparams = pltpu.CompilerParams(
        dimension_semantics=("parallel",),
        vmem_limit_bytes=48 * 1024 * 1024)

    k1 = functools.partial(_conv_relu_stats_kernel, nb=NB1, cin=Cin, kh=KH,
                           kw=KW, h=H, w=W, wp=Wp, span=span, ow=OW)
    y_flat, part_s, part_q = pl.pallas_call(
        k1,
        out_shape=(jax.ShapeDtypeStruct((N, Cout, OH * OW), jnp.bfloat16),
                   jax.ShapeDtypeStruct((N, Cout, 1), jnp.float32),
                   jax.ShapeDtypeStruct((N, Cout, 1), jnp.float32)),
        grid_spec=pltpu.PrefetchScalarGridSpec(
            num_scalar_prefetch=0,
            grid=(N // NB1,),
            in_specs=[pl.BlockSpec((NB1, Cin, H * W), lambda n: (n, 0, 0)),
                      pl.BlockSpec((Cout, K), lambda n: (0, 0)),
                      pl.BlockSpec((Cout, 1), lambda n: (0, 0))],
            out_specs=(pl.BlockSpec((NB1, Cout, OH * OW), lambda n: (n, 0, 0)),
                       pl.BlockSpec((NB1, Cout, 1), lambda n: (n, 0, 0)),
                       pl.BlockSpec((NB1, Cout, 1), lambda n: (n, 0, 0))),
            scratch_shapes=[pltpu.VMEM((Cin, L), jnp.bfloat16),
                            pltpu.VMEM((K, span), jnp.bfloat16)]),
        compiler_params=cparams,
    )(xflat, w2, b2)

    k2 = functools.partial(_bn_apply_kernel, m=float(M), eps=eps, nb=NB2)
    out_flat = pl.pallas_call(
        k2,
        out_shape=jax.ShapeDtypeStruct((N, Cout, OH * OW), jnp.float32),
        grid_spec=pltpu.PrefetchScalarGridSpec(
            num_scalar_prefetch=0,
            grid=(N // NB2,),
            in_specs=[pl.BlockSpec((NB2, Cout, OH * OW), lambda n: (n, 0, 0)),
                      pl.BlockSpec((N, Cout, 1), lambda n: (0, 0, 0)),
                      pl.BlockSpec((N, Cout, 1), lambda n: (0, 0, 0)),
                      pl.BlockSpec((Cout, 1), lambda n: (0, 0)),
                      pl.BlockSpec((Cout, 1), lambda n: (0, 0))],
            out_specs=pl.BlockSpec((NB2, Cout, OH * OW), lambda n: (n, 0, 0))),
        compiler_params=cparams,
    )(y_flat, part_s, part_q, g2, be2)

    return out_flat.reshape(N, Cout, OH, OW)


# speedup vs baseline: 2.0681x; 1.0147x over previous
import functools

import jax
import jax.numpy as jnp
from jax.experimental import pallas as pl
from jax.experimental.pallas import tpu as pltpu


def _fused_kernel(x_ref, w_ref, b_ref, g_ref, be_ref, o_ref,
                  img_ref, p_ref, y_ref, s_ref, q_ref, *,
                  nb1, nb2, c1, cin, kh, kw, h, w, wp, span, ow, m, eps):
    step = pl.program_id(0)
    oh = span // wp

    @pl.when(step < c1)
    def conv_phase():
        @pl.when(step == 0)
        def _init():
            s_ref[...] = jnp.zeros_like(s_ref)
            q_ref[...] = jnp.zeros_like(q_ref)

        for b in range(nb1):
            img_ref[...] = jnp.zeros_like(img_ref)
            xin = x_ref[b]
            for r in range(h):
                img_ref[:, (r + 1) * wp + 1:(r + 1) * wp + 1 + w] = (
                    xin[:, r * w:(r + 1) * w].astype(jnp.bfloat16))

            img = img_ref[...]
            for t in range(kh * kw):
                ki, kj = divmod(t, kw)
                start = ki * wp + kj
                p_ref[t * cin:(t + 1) * cin, :] = img[:, start:start + span]

            acc = jax.lax.dot_general(
                w_ref[...], p_ref[...],
                dimension_numbers=(((1,), (0,)), ((), ())),
                preferred_element_type=jnp.float32)
            y = jnp.maximum(acc + b_ref[...], 0.0)

            yc = jnp.concatenate(
                [y[:, r * wp:r * wp + ow] for r in range(oh)], axis=1)
            y_ref[step * nb1 + b] = yc.astype(jnp.bfloat16)

            s_ref[...] += jnp.sum(yc, axis=1, keepdims=True)
            q_ref[...] += jnp.sum(yc * yc, axis=1, keepdims=True)

    @pl.when(step >= c1)
    def apply_phase():
        mean = s_ref[...] / m
        var = jnp.maximum(q_ref[...] / m - mean * mean, 0.0)
        inv = jax.lax.rsqrt(var + eps)
        scale = g_ref[...] * inv
        shift = be_ref[...] - mean * scale
        base = (step - c1) * nb2
        for b in range(nb2):
            yv = y_ref[base + b].astype(jnp.float32)
            o_ref[b] = yv * scale + shift


def kernel(x, weight, bias, gamma, beta):
    padding, eps = 1, 1e-5
    N, Cin, H, W = x.shape
    Cout, _, KH, KW = weight.shape

    OH = H + 2 * padding - KH + 1
    OW = W + 2 * padding - KW + 1
    Hp, Wp = H + 2 * padding, W + 2 * padding
    span = OH * Wp
    M = N * OH * OW
    K = KH * KW * Cin
    L = max((KH - 1) * Wp + (KW - 1) + span, Hp * Wp)

    NB1 = 2 if N % 2 == 0 else 1
    NB2 = 4 if N % 4 == 0 else 1
    C1 = N // NB1
    C2 = N // NB2

    xflat = x.reshape(N, Cin, H * W)

    w2 = jnp.transpose(weight, (0, 2, 3, 1)).reshape(Cout, K).astype(jnp.bfloat16)
    b2 = bias.reshape(Cout, 1).astype(jnp.float32)
    g2 = gamma.reshape(Cout, 1).astype(jnp.float32)
    be2 = beta.reshape(Cout, 1).astype(jnp.float32)

    body = functools.partial(_fused_kernel, nb1=NB1, nb2=NB2, c1=C1, cin=Cin,
                             kh=KH, kw=KW, h=H, w=W, wp=Wp, span=span, ow=OW,
                             m=float(M), eps=eps)
    out_flat = pl.pallas_call(
        body,
        out_shape=jax.ShapeDtypeStruct((N, Cout, OH * OW), jnp.float32),
        grid_spec=pltpu.PrefetchScalarGridSpec(
            num_scalar_prefetch=0,
            grid=(C1 + C2,),
            in_specs=[
                pl.BlockSpec((NB1, Cin, H * W),
                             lambda s: (jnp.minimum(s, C1 - 1), 0, 0)),
                pl.BlockSpec((Cout, K), lambda s: (0, 0)),
                pl.BlockSpec((Cout, 1), lambda s: (0, 0)),
                pl.BlockSpec((Cout, 1), lambda s: (0, 0)),
                pl.BlockSpec((Cout, 1), lambda s: (0, 0))],
            out_specs=pl.BlockSpec((NB2, Cout, OH * OW),
                                   lambda s: (jnp.maximum(s - C1, 0), 0, 0)),
            scratch_shapes=[pltpu.VMEM((Cin, L), jnp.bfloat16),
                            pltpu.VMEM((K, span), jnp.bfloat16),
                            pltpu.VMEM((N, Cout, OH * OW), jnp.bfloat16),
                            pltpu.VMEM((Cout, 1), jnp.float32),
                            pltpu.VMEM((Cout, 1), jnp.float32)]),
        compiler_params=pltpu.CompilerParams(
            dimension_semantics=("arbitrary",),
            vmem_limit_bytes=60 * 1024 * 1024),
    )(xflat, w2, b2, g2, be2)

    return out_flat.reshape(N, Cout, OH, OW)


# speedup vs baseline: 2.2567x; 1.0912x over previous
import functools

import jax
import jax.numpy as jnp
from jax.experimental import pallas as pl
from jax.experimental.pallas import tpu as pltpu


def _fused_kernel(x_ref, w_ref, b_ref, g_ref, be_ref, mm_ref, mp_ref, o_ref,
                  xe_ref, p_ref, y_ref, s_ref, q_ref, *,
                  nb1, nb2, c1, cin, kh, kw, h, w, m, eps):
    step = pl.program_id(0)
    hw = h * w
    border = w + 1

    @pl.when(step < c1)
    def conv_phase():
        @pl.when(step == 0)
        def _init():
            xe_ref[...] = jnp.zeros_like(xe_ref)
            s_ref[...] = jnp.zeros_like(s_ref)
            q_ref[...] = jnp.zeros_like(q_ref)

        for b in range(nb1):
            xe_ref[:, border:border + hw] = x_ref[b].astype(jnp.bfloat16)

            for t in range(kh * kw):
                ki, kj = divmod(t, kw)
                st = border + (ki - 1) * w + (kj - 1)
                win = xe_ref[:, st:st + hw]
                if kj == 0:
                    win = win * mm_ref[...]
                elif kj == kw - 1:
                    win = win * mp_ref[...]
                p_ref[t * cin:(t + 1) * cin, :] = win

            acc = jax.lax.dot_general(
                w_ref[...], p_ref[...],
                dimension_numbers=(((1,), (0,)), ((), ())),
                preferred_element_type=jnp.float32)
            y = jnp.maximum(acc + b_ref[...], 0.0)
            y_ref[step * nb1 + b] = y.astype(jnp.bfloat16)

            s_ref[...] += jnp.sum(y, axis=1, keepdims=True)
            q_ref[...] += jnp.sum(y * y, axis=1, keepdims=True)

    @pl.when(step >= c1)
    def apply_phase():
        mean = s_ref[...] / m
        var = jnp.maximum(q_ref[...] / m - mean * mean, 0.0)
        inv = jax.lax.rsqrt(var + eps)
        scale = g_ref[...] * inv
        shift = be_ref[...] - mean * scale
        base = (step - c1) * nb2
        for b in range(nb2):
            yv = y_ref[base + b].astype(jnp.float32)
            o_ref[b] = yv * scale + shift


def kernel(x, weight, bias, gamma, beta):
    padding, eps = 1, 1e-5
    N, Cin, H, W = x.shape
    Cout, _, KH, KW = weight.shape
    assert KH == 3 and KW == 3 and padding == 1

    M = N * H * W
    K = KH * KW * Cin
    HW = H * W

    NB1 = 2 if N % 2 == 0 else 1
    NB2 = 4 if N % 4 == 0 else 1
    C1 = N // NB1
    C2 = N // NB2

    xflat = x.reshape(N, Cin, HW)

    w2 = jnp.transpose(weight, (0, 2, 3, 1)).reshape(Cout, K).astype(jnp.bfloat16)
    b2 = bias.reshape(Cout, 1).astype(jnp.float32)
    g2 = gamma.reshape(Cout, 1).astype(jnp.float32)
    be2 = beta.reshape(Cout, 1).astype(jnp.float32)

    lane = jnp.arange(HW, dtype=jnp.int32) % W
    maskm = (lane != 0).astype(jnp.bfloat16).reshape(1, HW)
    maskp = (lane != W - 1).astype(jnp.bfloat16).reshape(1, HW)

    body = functools.partial(_fused_kernel, nb1=NB1, nb2=NB2, c1=C1, cin=Cin,
                             kh=KH, kw=KW, h=H, w=W, m=float(M), eps=eps)
    out_flat = pl.pallas_call(
        body,
        out_shape=jax.ShapeDtypeStruct((N, Cout, HW), jnp.float32),
        grid_spec=pltpu.PrefetchScalarGridSpec(
            num_scalar_prefetch=0,
            grid=(C1 + C2,),
            in_specs=[
                pl.BlockSpec((NB1, Cin, HW),
                             lambda s: (jnp.minimum(s, C1 - 1), 0, 0)),
                pl.BlockSpec((Cout, K), lambda s: (0, 0)),
                pl.BlockSpec((Cout, 1), lambda s: (0, 0)),
                pl.BlockSpec((Cout, 1), lambda s: (0, 0)),
                pl.BlockSpec((Cout, 1), lambda s: (0, 0)),
                pl.BlockSpec((1, HW), lambda s: (0, 0)),
                pl.BlockSpec((1, HW), lambda s: (0, 0))],
            out_specs=pl.BlockSpec((NB2, Cout, HW),
                                   lambda s: (jnp.maximum(s - C1, 0), 0, 0)),
            scratch_shapes=[pltpu.VMEM((Cin, HW + 2 * (W + 1)), jnp.bfloat16),
                            pltpu.VMEM((K, HW), jnp.bfloat16),
                            pltpu.VMEM((N, Cout, HW), jnp.bfloat16),
                            pltpu.VMEM((Cout, 1), jnp.float32),
                            pltpu.VMEM((Cout, 1), jnp.float32)]),
        compiler_params=pltpu.CompilerParams(
            dimension_semantics=("arbitrary",),
            vmem_limit_bytes=60 * 1024 * 1024),
    )(xflat, w2, b2, g2, be2, maskm, maskp)

    return out_flat.reshape(N, Cout, H, W)


# speedup vs baseline: 2.3317x; 1.0332x over previous
import functools

import jax
import jax.numpy as jnp
from jax.experimental import pallas as pl
from jax.experimental.pallas import tpu as pltpu


def _fused_kernel(x_ref, w_ref, b_ref, g_ref, be_ref, mm_ref, mp_ref, o_ref,
                  xe_ref, xm_ref, xp_ref, p_ref, y_ref, s_ref, q_ref, *,
                  nb1, nb2, c1, cin, kh, kw, h, w, m, eps):
    step = pl.program_id(0)
    hw = h * w
    border = w + 1

    @pl.when(step < c1)
    def conv_phase():
        @pl.when(step == 0)
        def _init():
            xe_ref[...] = jnp.zeros_like(xe_ref)
            xm_ref[...] = jnp.zeros_like(xm_ref)
            xp_ref[...] = jnp.zeros_like(xp_ref)
            s_ref[...] = jnp.zeros_like(s_ref)
            q_ref[...] = jnp.zeros_like(q_ref)

        for b in range(nb1):
            xb = x_ref[b].astype(jnp.bfloat16)
            xe_ref[:, border:border + hw] = xb
            xm_ref[:, border:border + hw] = xb * mm_ref[...]
            xp_ref[:, border:border + hw] = xb * mp_ref[...]

            for t in range(kh * kw):
                ki, kj = divmod(t, kw)
                st = border + (ki - 1) * w + (kj - 1)
                src = xm_ref if kj == 0 else (xp_ref if kj == kw - 1 else xe_ref)
                p_ref[t * cin:(t + 1) * cin, :] = src[:, st:st + hw]

            acc = jax.lax.dot_general(
                w_ref[...], p_ref[...],
                dimension_numbers=(((1,), (0,)), ((), ())),
                preferred_element_type=jnp.float32)
            y = jnp.maximum(acc + b_ref[...], 0.0)
            y_ref[step * nb1 + b] = y.astype(jnp.bfloat16)

            s_ref[...] += jnp.sum(y, axis=1, keepdims=True)
            q_ref[...] += jnp.sum(y * y, axis=1, keepdims=True)

    @pl.when(step >= c1)
    def apply_phase():
        mean = s_ref[...] / m
        var = jnp.maximum(q_ref[...] / m - mean * mean, 0.0)
        inv = jax.lax.rsqrt(var + eps)
        scale = g_ref[...] * inv
        shift = be_ref[...] - mean * scale
        base = (step - c1) * nb2
        for b in range(nb2):
            yv = y_ref[base + b].astype(jnp.float32)
            o_ref[b] = yv * scale + shift


def kernel(x, weight, bias, gamma, beta):
    padding, eps = 1, 1e-5
    N, Cin, H, W = x.shape
    Cout, _, KH, KW = weight.shape
    assert KH == 3 and KW == 3 and padding == 1

    M = N * H * W
    K = KH * KW * Cin
    HW = H * W

    NB1 = 2 if N % 2 == 0 else 1
    NB2 = 4 if N % 4 == 0 else 1
    C1 = N // NB1
    C2 = N // NB2

    xflat = x.reshape(N, Cin, HW)

    w2 = jnp.transpose(weight, (0, 2, 3, 1)).reshape(Cout, K).astype(jnp.bfloat16)
    b2 = bias.reshape(Cout, 1).astype(jnp.float32)
    g2 = gamma.reshape(Cout, 1).astype(jnp.float32)
    be2 = beta.reshape(Cout, 1).astype(jnp.float32)

    lane = jnp.arange(HW, dtype=jnp.int32) % W
    maskm = (lane != W - 1).astype(jnp.bfloat16).reshape(1, HW)
    maskp = (lane != 0).astype(jnp.bfloat16).reshape(1, HW)

    body = functools.partial(_fused_kernel, nb1=NB1, nb2=NB2, c1=C1, cin=Cin,
                             kh=KH, kw=KW, h=H, w=W, m=float(M), eps=eps)
    out_flat = pl.pallas_call(
        body,
        out_shape=jax.ShapeDtypeStruct((N, Cout, HW), jnp.float32),
        grid_spec=pltpu.PrefetchScalarGridSpec(
            num_scalar_prefetch=0,
            grid=(C1 + C2,),
            in_specs=[
                pl.BlockSpec((NB1, Cin, HW),
                             lambda s: (jnp.minimum(s, C1 - 1), 0, 0)),
                pl.BlockSpec((Cout, K), lambda s: (0, 0)),
                pl.BlockSpec((Cout, 1), lambda s: (0, 0)),
                pl.BlockSpec((Cout, 1), lambda s: (0, 0)),
                pl.BlockSpec((Cout, 1), lambda s: (0, 0)),
                pl.BlockSpec((1, HW), lambda s: (0, 0)),
                pl.BlockSpec((1, HW), lambda s: (0, 0))],
            out_specs=pl.BlockSpec((NB2, Cout, HW),
                                   lambda s: (jnp.maximum(s - C1, 0), 0, 0)),
            scratch_shapes=[pltpu.VMEM((Cin, HW + 2 * (W + 1)), jnp.bfloat16),
                            pltpu.VMEM((Cin, HW + 2 * (W + 1)), jnp.bfloat16),
                            pltpu.VMEM((Cin, HW + 2 * (W + 1)), jnp.bfloat16),
                            pltpu.VMEM((K, HW), jnp.bfloat16),
                            pltpu.VMEM((N, Cout, HW), jnp.bfloat16),
                            pltpu.VMEM((Cout, 1), jnp.float32),
                            pltpu.VMEM((Cout, 1), jnp.float32)]),
        compiler_params=pltpu.CompilerParams(
            dimension_semantics=("arbitrary",),
            vmem_limit_bytes=60 * 1024 * 1024),
    )(xflat, w2, b2, g2, be2, maskm, maskp)

    return out_flat.reshape(N, Cout, H, W)


# speedup vs baseline: 2.5037x; 1.0738x over previous
import functools

import jax
import jax.numpy as jnp
from jax.experimental import pallas as pl
from jax.experimental.pallas import tpu as pltpu


def _fused_kernel(x_ref, w_ref, b_ref, g_ref, be_ref, mm_ref, mp_ref, o_ref,
                  xe_ref, xm_ref, xp_ref, p_ref, y_ref, s_ref, q_ref, *,
                  nb1, nb2, c1, cin, kh, kw, h, w, m, eps):
    step = pl.program_id(0)
    hw = h * w
    border = 128

    @pl.when(step < c1)
    def conv_phase():
        @pl.when(step == 0)
        def _init():
            xe_ref[...] = jnp.zeros_like(xe_ref)
            xm_ref[...] = jnp.zeros_like(xm_ref)
            xp_ref[...] = jnp.zeros_like(xp_ref)
            s_ref[...] = jnp.zeros_like(s_ref)
            q_ref[...] = jnp.zeros_like(q_ref)

        for b in range(nb1):
            xb = x_ref[b].astype(jnp.bfloat16)
            xe_ref[:, border:border + hw] = xb
            xm_ref[:, border:border + hw] = xb * mm_ref[...]
            xp_ref[:, border:border + hw] = xb * mp_ref[...]

            for t in range(kh * kw):
                ki, kj = divmod(t, kw)
                st = border + (ki - 1) * w + (kj - 1)
                src = xm_ref if kj == 0 else (xp_ref if kj == kw - 1 else xe_ref)
                p_ref[t * cin:(t + 1) * cin, :] = src[:, st:st + hw]

            acc = jax.lax.dot_general(
                w_ref[...], p_ref[...],
                dimension_numbers=(((1,), (0,)), ((), ())),
                preferred_element_type=jnp.float32)
            y = jnp.maximum(acc + b_ref[...], 0.0)
            y_ref[step * nb1 + b] = y.astype(jnp.bfloat16)

            s_ref[...] += jnp.sum(y, axis=1, keepdims=True)
            q_ref[...] += jnp.sum(y * y, axis=1, keepdims=True)

    @pl.when(step >= c1)
    def apply_phase():
        mean = s_ref[...] / m
        var = jnp.maximum(q_ref[...] / m - mean * mean, 0.0)
        inv = jax.lax.rsqrt(var + eps)
        scale = g_ref[...] * inv
        shift = be_ref[...] - mean * scale
        base = (step - c1) * nb2
        for b in range(nb2):
            yv = y_ref[base + b].astype(jnp.float32)
            o_ref[b] = yv * scale + shift


def kernel(x, weight, bias, gamma, beta):
    padding, eps = 1, 1e-5
    N, Cin, H, W = x.shape
    Cout, _, KH, KW = weight.shape
    assert KH == 3 and KW == 3 and padding == 1

    M = N * H * W
    K = KH * KW * Cin
    HW = H * W

    NB1 = 4 if N % 4 == 0 else 1
    NB2 = 8 if N % 8 == 0 else 1
    C1 = N // NB1
    C2 = N // NB2

    xflat = x.reshape(N, Cin, HW)

    w2 = jnp.transpose(weight, (0, 2, 3, 1)).reshape(Cout, K).astype(jnp.bfloat16)
    b2 = bias.reshape(Cout, 1).astype(jnp.float32)
    g2 = gamma.reshape(Cout, 1).astype(jnp.float32)
    be2 = beta.reshape(Cout, 1).astype(jnp.float32)

    lane = jnp.arange(HW, dtype=jnp.int32) % W
    maskm = (lane != W - 1).astype(jnp.bfloat16).reshape(1, HW)
    maskp = (lane != 0).astype(jnp.bfloat16).reshape(1, HW)

    body = functools.partial(_fused_kernel, nb1=NB1, nb2=NB2, c1=C1, cin=Cin,
                             kh=KH, kw=KW, h=H, w=W, m=float(M), eps=eps)
    out_flat = pl.pallas_call(
        body,
        out_shape=jax.ShapeDtypeStruct((N, Cout, HW), jnp.float32),
        grid_spec=pltpu.PrefetchScalarGridSpec(
            num_scalar_prefetch=0,
            grid=(C1 + C2,),
            in_specs=[
                pl.BlockSpec((NB1, Cin, HW),
                             lambda s: (jnp.minimum(s, C1 - 1), 0, 0)),
                pl.BlockSpec((Cout, K), lambda s: (0, 0)),
                pl.BlockSpec((Cout, 1), lambda s: (0, 0)),
                pl.BlockSpec((Cout, 1), lambda s: (0, 0)),
                pl.BlockSpec((Cout, 1), lambda s: (0, 0)),
                pl.BlockSpec((1, HW), lambda s: (0, 0)),
                pl.BlockSpec((1, HW), lambda s: (0, 0))],
            out_specs=pl.BlockSpec((NB2, Cout, HW),
                                   lambda s: (jnp.maximum(s - C1, 0), 0, 0)),
            scratch_shapes=[pltpu.VMEM((Cin, HW + 256 + 2 * (W + 1)), jnp.bfloat16),
                            pltpu.VMEM((Cin, HW + 256 + 2 * (W + 1)), jnp.bfloat16),
                            pltpu.VMEM((Cin, HW + 256 + 2 * (W + 1)), jnp.bfloat16),
                            pltpu.VMEM((K, HW), jnp.bfloat16),
                            pltpu.VMEM((N, Cout, HW), jnp.bfloat16),
                            pltpu.VMEM((Cout, 1), jnp.float32),
                            pltpu.VMEM((Cout, 1), jnp.float32)]),
        compiler_params=pltpu.CompilerParams(
            dimension_semantics=("arbitrary",),
            vmem_limit_bytes=60 * 1024 * 1024),
    )(xflat, w2, b2, g2, be2, maskm, maskp)

    return out_flat.reshape(N, Cout, H, W)
